# baseline (speedup 1.0000x reference)
import functools
import math

import jax
import jax.numpy as jnp
import numpy as np
from jax import lax
from jax.experimental import pallas as pl
from jax.experimental.pallas import tpu as pltpu
from jax.experimental.pallas import tpu_sc as plsc

PATCH_LEN = 16
STRIDE = 8
SEQ_LEN = 512
D_MODEL = 512
HIDDEN = 256
PATCH_NUM = 64
NWIN = 505
SEG = 640
SB = 8
MLANE = SB * SEG
NW = 32


def _pos_emb(n, d_model):
    position = np.arange(n, dtype=np.float32)[:, None]
    div_term = np.exp(np.arange(0, d_model, 2, dtype=np.float32) * -(math.log(10000.0) / d_model))
    pe = np.zeros((n, d_model), dtype=np.float32)
    pe[:, 0::2] = np.sin(position * div_term)
    pe[:, 1::2] = np.cos(position * div_term)
    return pe


NT = (((1,), (1,)), ((), ()))


def _score_body(x_ref, w1t_ref, b1c_ref, w2t_ref, b2c_ref, idxf_ref, ms_ref):
    f32 = jnp.float32
    xflat = x_ref[...]

    pieces = [xflat]
    for p in range(1, PATCH_LEN):
        pieces.append(jnp.concatenate([xflat[:, p:], xflat[:, :p]], axis=1))
    xr_all = jnp.concatenate(pieces, axis=0)

    pre1 = jnp.dot(w1t_ref[...], xr_all, preferred_element_type=f32)
    hid = jnp.maximum(pre1 + b1c_ref[...], 0.0)

    scr_rows = []
    for s in range(SB):
        h_s = hid[:, s * SEG:(s + 1) * SEG]
        scr_rows.append(jnp.dot(w2t_ref[...], h_s, preferred_element_type=f32)
                        + b2c_ref[...])
    scr = jnp.concatenate(scr_rows, axis=0)

    iota_u = lax.broadcasted_iota(jnp.int32, (SB * PATCH_NUM, SEG), 1)
    scr = jnp.where(iota_u < NWIN, scr, -1e30)

    maxv = jnp.max(scr, axis=1, keepdims=True)
    eq = scr == maxv
    idx = jnp.min(jnp.where(eq, iota_u, SEG), axis=1, keepdims=True)
    idxf_ref[...] = idx.astype(f32)
    ms_ref[...] = (maxv != 0.0).astype(f32)


def _make_sc_gather(S):
    mesh = plsc.VectorSubcoreMesh(core_axis_name="c", subcore_axis_name="s")
    per = (S + NW - 1) // NW
    BF = 7 if per % 7 == 0 else 1
    nbatch = per // BF
    POUT = PATCH_NUM * PATCH_LEN

    @functools.partial(
        pl.kernel, mesh=mesh,
        compiler_params=pltpu.CompilerParams(needs_layout_passes=False),
        out_type=[jax.ShapeDtypeStruct((S * POUT,), jnp.float32),
                  jax.ShapeDtypeStruct((S * POUT,), jnp.float32)],
        scratch_types=[
            pltpu.VMEM((BF * SEG,), jnp.float32),
            pltpu.VMEM((BF * PATCH_NUM,), jnp.float32),
            pltpu.VMEM((BF * PATCH_NUM,), jnp.float32),
            pltpu.VMEM((BF * POUT,), jnp.float32),
            pltpu.VMEM((BF * POUT,), jnp.float32),
        ],
    )
    def gather_k(x_hbm, idxf_hbm, ms_hbm, sel_hbm, org_hbm,
                 xrow_v, idxf_v, ms_v, sel_v, org_v):
        wid = lax.axis_index("s") * 2 + lax.axis_index("c")
        base = wid * per
        iota16 = lax.iota(jnp.int32, 16)

        def batch_body(kb, _):
            s0 = base + kb * BF

            @pl.when(s0 < S)
            def _():
                pltpu.sync_copy(x_hbm.at[pl.ds(s0 * SEG, BF * SEG)], xrow_v)
                pltpu.sync_copy(idxf_hbm.at[pl.ds(s0 * PATCH_NUM, BF * PATCH_NUM)], idxf_v)
                pltpu.sync_copy(ms_hbm.at[pl.ds(s0 * PATCH_NUM, BF * PATCH_NUM)], ms_v)
                for r in range(BF):
                    for t in range(4):
                        idxc = idxf_v[pl.ds(r * PATCH_NUM + 16 * t, 16)].astype(jnp.int32)
                        msc = ms_v[pl.ds(r * PATCH_NUM + 16 * t, 16)]
                        lanes = iota16 * 16 + 256 * t + r * POUT
                        for p in range(16):
                            vals = plsc.load_gather(xrow_v, [idxc + p + r * SEG])
                            plsc.store_scatter(sel_v, [lanes + p], vals * msc)
                            ovals = plsc.load_gather(
                                xrow_v, [8 * (iota16 + 16 * t) + p + r * SEG])
                            plsc.store_scatter(org_v, [lanes + p], ovals)
                pltpu.sync_copy(sel_v, sel_hbm.at[pl.ds(s0 * POUT, BF * POUT)])
                pltpu.sync_copy(org_v, org_hbm.at[pl.ds(s0 * POUT, BF * POUT)])

            return 0

        lax.fori_loop(0, nbatch, batch_body, 0)

    return gather_k


def _embed_body(sel_ref, org_ref, w1s_ref, b1s_ref, w2s_ref, b2s_ref,
                wreg_ref, wirr_ref, alpha_ref, pe_ref, out_ref):
    f32 = jnp.float32
    sel = sel_ref[...]
    org = org_ref[...]

    hid_s = jnp.maximum(jnp.dot(sel, w1s_ref[...], preferred_element_type=f32)
                        + b1s_ref[...], 0.0)
    shc = jnp.dot(hid_s, w2s_ref[...], preferred_element_type=f32) + b2s_ref[...]

    shm = shc.reshape(SB, PATCH_NUM)
    sh_big = jnp.broadcast_to(shm[:, None, :],
                              (SB, PATCH_NUM, PATCH_NUM)).reshape(SB * PATCH_NUM, PATCH_NUM)
    iota_r = lax.broadcasted_iota(jnp.int32, (SB * PATCH_NUM, 1), 0)
    jmod_col = iota_r & (PATCH_NUM - 1)
    iota_i = lax.broadcasted_iota(jnp.int32, (SB * PATCH_NUM, PATCH_NUM), 1)
    lt = shc < sh_big
    eq2 = shc == sh_big
    cmp2 = jnp.where(lt | (eq2 & (jmod_col < iota_i)), 1.0, 0.0)
    rank_sb = jnp.sum(cmp2.reshape(SB, PATCH_NUM, PATCH_NUM), axis=1)
    nz_sb = (shm != 0.0).astype(f32)
    rank_big = jnp.broadcast_to(rank_sb[:, None, :],
                                (SB, PATCH_NUM, PATCH_NUM)).reshape(SB * PATCH_NUM, PATCH_NUM)
    nz_big = jnp.broadcast_to(nz_sb[:, None, :],
                              (SB, PATCH_NUM, PATCH_NUM)).reshape(SB * PATCH_NUM, PATCH_NUM)
    perm_big = jnp.where(rank_big == jmod_col.astype(f32), nz_big, 0.0)

    shuf_rows = []
    for s in range(SB):
        shuf_rows.append(jnp.dot(perm_big[s * PATCH_NUM:(s + 1) * PATCH_NUM],
                                 sel[s * PATCH_NUM:(s + 1) * PATCH_NUM],
                                 preferred_element_type=f32))
    shuf = jnp.concatenate(shuf_rows, axis=0)

    a = alpha_ref[0, 0]
    w = 1.0 / (1.0 + jnp.exp(-a))
    emb = (w * jnp.dot(org, wreg_ref[...], preferred_element_type=f32)
           + (1.0 - w) * jnp.dot(shuf, wirr_ref[...], preferred_element_type=f32))
    out_ref[...] = emb.reshape(SB, PATCH_NUM, D_MODEL) + pe_ref[...][None]


def _const_spec(shape):
    return pl.BlockSpec(shape, lambda i: (0,) * len(shape))


def kernel(x, W1_sel, b1_sel, W2_sel, b2_sel, W1_shf, b1_shf, W2_shf, b2_shf,
           W_reg, W_irr, alpha):
    b, c, L = x.shape
    S_all = b * c
    xf = x.reshape(S_all, L)
    x_pad_all = jnp.concatenate(
        [xf, jnp.repeat(xf[:, -1:], SEG - L, axis=1)], axis=1)

    pe = jnp.asarray(_pos_emb(PATCH_NUM, D_MODEL))

    NCH = 3 if (S_all % 3 == 0 and (S_all // 3) % SB == 0
                and ((S_all // 3 + NW - 1) // NW) % 7 == 0) else 1
    CS = S_all // NCH
    outs = []
    for ch in range(NCH):
        x_pad = lax.slice_in_dim(x_pad_all, ch * CS, (ch + 1) * CS, axis=0)
        outs.append(_run_chunk(x_pad, CS, W1_sel, b1_sel, W2_sel, b2_sel,
                               W1_shf, b1_shf, W2_shf, b2_shf,
                               W_reg, W_irr, alpha, pe))
    return outs[0] if NCH == 1 else jnp.concatenate(outs, axis=0)


def _run_chunk(x_pad, S, W1_sel, b1_sel, W2_sel, b2_sel, W1_shf, b1_shf,
               W2_shf, b2_shf, W_reg, W_irr, alpha, pe):
    x_flat = x_pad.reshape(1, S * SEG)

    idxf, ms = pl.pallas_call(
        _score_body,
        grid=(S // SB,),
        in_specs=[
            pl.BlockSpec((1, MLANE), lambda i: (0, i)),
            _const_spec((HIDDEN, PATCH_LEN)),
            _const_spec((HIDDEN, 1)),
            _const_spec((PATCH_NUM, HIDDEN)),
            _const_spec((PATCH_NUM, 1)),
        ],
        out_specs=[pl.BlockSpec((SB * PATCH_NUM, 1), lambda i: (i, 0)),
                   pl.BlockSpec((SB * PATCH_NUM, 1), lambda i: (i, 0))],
        out_shape=[jax.ShapeDtypeStruct((S * PATCH_NUM, 1), jnp.float32),
                   jax.ShapeDtypeStruct((S * PATCH_NUM, 1), jnp.float32)],
        compiler_params=pltpu.CompilerParams(
            dimension_semantics=("arbitrary",),
        ),
    )(x_flat, W1_sel.T, b1_sel.reshape(HIDDEN, 1), W2_sel.T,
      b2_sel.reshape(PATCH_NUM, 1))

    sel_flat, org_flat = _make_sc_gather(S)(
        x_pad.reshape(S * SEG), idxf.reshape(S * PATCH_NUM), ms.reshape(S * PATCH_NUM))

    out = pl.pallas_call(
        _embed_body,
        grid=(S // SB,),
        in_specs=[
            pl.BlockSpec((SB * PATCH_NUM, PATCH_LEN), lambda i: (i, 0)),
            pl.BlockSpec((SB * PATCH_NUM, PATCH_LEN), lambda i: (i, 0)),
            _const_spec((PATCH_LEN, HIDDEN)),
            _const_spec((1, HIDDEN)),
            _const_spec((HIDDEN, 1)),
            _const_spec((1, 1)),
            _const_spec((PATCH_LEN, D_MODEL)),
            _const_spec((PATCH_LEN, D_MODEL)),
            _const_spec((1, 1)),
            _const_spec((PATCH_NUM, D_MODEL)),
        ],
        out_specs=pl.BlockSpec((SB, PATCH_NUM, D_MODEL), lambda i: (i, 0, 0)),
        out_shape=jax.ShapeDtypeStruct((S, PATCH_NUM, D_MODEL), jnp.float32),
        compiler_params=pltpu.CompilerParams(
            dimension_semantics=("arbitrary",),
        ),
    )(sel_flat.reshape(S * PATCH_NUM, PATCH_LEN),
      org_flat.reshape(S * PATCH_NUM, PATCH_LEN),
      W1_shf, b1_shf.reshape(1, HIDDEN), W2_shf, b2_shf.reshape(1, 1),
      W_reg, W_irr, alpha.reshape(1, 1), pe)
    return out


# speedup vs baseline: 1.1817x; 1.1817x over previous
import functools
import math

import jax
import jax.numpy as jnp
import numpy as np
from jax import lax
from jax.experimental import pallas as pl
from jax.experimental.pallas import tpu as pltpu
from jax.experimental.pallas import tpu_sc as plsc

PATCH_LEN = 16
STRIDE = 8
SEQ_LEN = 512
D_MODEL = 512
HIDDEN = 256
PATCH_NUM = 64
NWIN = 505
SEG = 640
SB = 8
MLANE = SB * SEG
NW = 32


def _pos_emb(n, d_model):
    position = np.arange(n, dtype=np.float32)[:, None]
    div_term = np.exp(np.arange(0, d_model, 2, dtype=np.float32) * -(math.log(10000.0) / d_model))
    pe = np.zeros((n, d_model), dtype=np.float32)
    pe[:, 0::2] = np.sin(position * div_term)
    pe[:, 1::2] = np.cos(position * div_term)
    return pe


NT = (((1,), (1,)), ((), ()))


def _score_body(x_ref, w1t_ref, b1c_ref, w2t_ref, b2c_ref, idxf_ref, ms_ref,
                org_ref):
    f32 = jnp.float32
    xflat = x_ref[...]

    pieces = [xflat]
    for p in range(1, PATCH_LEN):
        pieces.append(jnp.concatenate([xflat[:, p:], xflat[:, :p]], axis=1))
    xr_all = jnp.concatenate(pieces, axis=0)

    pre1 = jnp.dot(w1t_ref[...], xr_all, preferred_element_type=f32)
    hid = jnp.maximum(pre1 + b1c_ref[...], 0.0)

    scr_rows = []
    for s in range(SB):
        h_s = hid[:, s * SEG:(s + 1) * SEG]
        scr_rows.append(jnp.dot(w2t_ref[...], h_s, preferred_element_type=f32)
                        + b2c_ref[...])
    scr = jnp.concatenate(scr_rows, axis=0)

    iota_u = lax.broadcasted_iota(jnp.int32, (SB * PATCH_NUM, SEG), 1)
    scr = jnp.where(iota_u < NWIN, scr, -1e30)

    maxv = jnp.max(scr, axis=1, keepdims=True)
    eq = scr == maxv
    idx = jnp.min(jnp.where(eq, iota_u, SEG), axis=1, keepdims=True)
    idxf_ref[...] = idx.astype(f32)
    ms_ref[...] = (maxv != 0.0).astype(f32)

    iota_j = lax.broadcasted_iota(jnp.int32, (PATCH_NUM, SEG), 0)
    iota_u1 = lax.broadcasted_iota(jnp.int32, (PATCH_NUM, SEG), 1)
    e_org = (iota_u1 == 8 * iota_j).astype(f32)
    org_rows = []
    for s in range(SB):
        xr_s = xr_all[:, s * SEG:(s + 1) * SEG]
        org_rows.append(lax.dot_general(e_org, xr_s, NT, preferred_element_type=f32))
    org_ref[...] = jnp.concatenate(org_rows, axis=0)


def _make_sc_gather(S):
    mesh = plsc.VectorSubcoreMesh(core_axis_name="c", subcore_axis_name="s")
    per = (S + NW - 1) // NW
    BF = 7 if per % 7 == 0 else 1
    nbatch = per // BF
    POUT = PATCH_NUM * PATCH_LEN

    @functools.partial(
        pl.kernel, mesh=mesh,
        compiler_params=pltpu.CompilerParams(needs_layout_passes=False),
        out_type=jax.ShapeDtypeStruct((S * POUT,), jnp.float32),
        scratch_types=[
            pltpu.VMEM((BF * SEG,), jnp.float32),
            pltpu.VMEM((BF * PATCH_NUM,), jnp.float32),
            pltpu.VMEM((BF * PATCH_NUM,), jnp.float32),
            pltpu.VMEM((BF * POUT,), jnp.float32),
        ],
    )
    def gather_k(x_hbm, idxf_hbm, ms_hbm, sel_hbm,
                 xrow_v, idxf_v, ms_v, sel_v):
        wid = lax.axis_index("s") * 2 + lax.axis_index("c")
        base = wid * per
        iota16 = lax.iota(jnp.int32, 16)

        def batch_body(kb, _):
            s0 = base + kb * BF

            @pl.when(s0 < S)
            def _():
                pltpu.sync_copy(x_hbm.at[pl.ds(s0 * SEG, BF * SEG)], xrow_v)
                pltpu.sync_copy(idxf_hbm.at[pl.ds(s0 * PATCH_NUM, BF * PATCH_NUM)], idxf_v)
                pltpu.sync_copy(ms_hbm.at[pl.ds(s0 * PATCH_NUM, BF * PATCH_NUM)], ms_v)
                for r in range(BF):
                    for t in range(4):
                        idxc = idxf_v[pl.ds(r * PATCH_NUM + 16 * t, 16)].astype(jnp.int32)
                        msc = ms_v[pl.ds(r * PATCH_NUM + 16 * t, 16)]
                        lanes = iota16 * 16 + 256 * t + r * POUT
                        for p in range(16):
                            vals = plsc.load_gather(xrow_v, [idxc + p + r * SEG])
                            plsc.store_scatter(sel_v, [lanes + p], vals * msc)
                pltpu.sync_copy(sel_v, sel_hbm.at[pl.ds(s0 * POUT, BF * POUT)])

            return 0

        lax.fori_loop(0, nbatch, batch_body, 0)

    return gather_k


def _embed_body(sel_ref, org_ref, w1s_ref, b1s_ref, w2s_ref, b2s_ref,
                wreg_ref, wirr_ref, alpha_ref, pe_ref, out_ref):
    f32 = jnp.float32
    sel = sel_ref[...]
    org = org_ref[...]

    hid_s = jnp.maximum(jnp.dot(sel, w1s_ref[...], preferred_element_type=f32)
                        + b1s_ref[...], 0.0)
    shc = jnp.dot(hid_s, w2s_ref[...], preferred_element_type=f32) + b2s_ref[...]

    shm = shc.reshape(SB, PATCH_NUM)
    sh_big = jnp.broadcast_to(shm[:, None, :],
                              (SB, PATCH_NUM, PATCH_NUM)).reshape(SB * PATCH_NUM, PATCH_NUM)
    iota_r = lax.broadcasted_iota(jnp.int32, (SB * PATCH_NUM, 1), 0)
    jmod_col = iota_r & (PATCH_NUM - 1)
    iota_i = lax.broadcasted_iota(jnp.int32, (SB * PATCH_NUM, PATCH_NUM), 1)
    lt = shc < sh_big
    eq2 = shc == sh_big
    cmp2 = jnp.where(lt | (eq2 & (jmod_col < iota_i)), 1.0, 0.0)
    rank_sb = jnp.sum(cmp2.reshape(SB, PATCH_NUM, PATCH_NUM), axis=1)
    nz_sb = (shm != 0.0).astype(f32)
    rank_big = jnp.broadcast_to(rank_sb[:, None, :],
                                (SB, PATCH_NUM, PATCH_NUM)).reshape(SB * PATCH_NUM, PATCH_NUM)
    nz_big = jnp.broadcast_to(nz_sb[:, None, :],
                              (SB, PATCH_NUM, PATCH_NUM)).reshape(SB * PATCH_NUM, PATCH_NUM)
    perm_big = jnp.where(rank_big == jmod_col.astype(f32), nz_big, 0.0)

    shuf_rows = []
    for s in range(SB):
        shuf_rows.append(jnp.dot(perm_big[s * PATCH_NUM:(s + 1) * PATCH_NUM],
                                 sel[s * PATCH_NUM:(s + 1) * PATCH_NUM],
                                 preferred_element_type=f32))
    shuf = jnp.concatenate(shuf_rows, axis=0)

    a = alpha_ref[0, 0]
    w = 1.0 / (1.0 + jnp.exp(-a))
    emb = (w * jnp.dot(org, wreg_ref[...], preferred_element_type=f32)
           + (1.0 - w) * jnp.dot(shuf, wirr_ref[...], preferred_element_type=f32))
    out_ref[...] = emb.reshape(SB, PATCH_NUM, D_MODEL) + pe_ref[...][None]


def _const_spec(shape):
    return pl.BlockSpec(shape, lambda i: (0,) * len(shape))


def kernel(x, W1_sel, b1_sel, W2_sel, b2_sel, W1_shf, b1_shf, W2_shf, b2_shf,
           W_reg, W_irr, alpha):
    b, c, L = x.shape
    S = b * c
    xf = x.reshape(S, L)
    x_pad = jnp.concatenate(
        [xf, jnp.repeat(xf[:, -1:], SEG - L, axis=1)], axis=1)

    pe = jnp.asarray(_pos_emb(PATCH_NUM, D_MODEL))
    x_flat = x_pad.reshape(1, S * SEG)

    idxf, ms, org = pl.pallas_call(
        _score_body,
        grid=(S // SB,),
        in_specs=[
            pl.BlockSpec((1, MLANE), lambda i: (0, i)),
            _const_spec((HIDDEN, PATCH_LEN)),
            _const_spec((HIDDEN, 1)),
            _const_spec((PATCH_NUM, HIDDEN)),
            _const_spec((PATCH_NUM, 1)),
        ],
        out_specs=[pl.BlockSpec((SB * PATCH_NUM, 1), lambda i: (i, 0)),
                   pl.BlockSpec((SB * PATCH_NUM, 1), lambda i: (i, 0)),
                   pl.BlockSpec((SB * PATCH_NUM, PATCH_LEN), lambda i: (i, 0))],
        out_shape=[jax.ShapeDtypeStruct((S * PATCH_NUM, 1), jnp.float32),
                   jax.ShapeDtypeStruct((S * PATCH_NUM, 1), jnp.float32),
                   jax.ShapeDtypeStruct((S * PATCH_NUM, PATCH_LEN), jnp.float32)],
        compiler_params=pltpu.CompilerParams(
            dimension_semantics=("arbitrary",),
        ),
    )(x_flat, W1_sel.T, b1_sel.reshape(HIDDEN, 1), W2_sel.T,
      b2_sel.reshape(PATCH_NUM, 1))

    sel_flat = _make_sc_gather(S)(
        x_pad.reshape(S * SEG), idxf.reshape(S * PATCH_NUM), ms.reshape(S * PATCH_NUM))

    out = pl.pallas_call(
        _embed_body,
        grid=(S // SB,),
        in_specs=[
            pl.BlockSpec((SB * PATCH_NUM, PATCH_LEN), lambda i: (i, 0)),
            pl.BlockSpec((SB * PATCH_NUM, PATCH_LEN), lambda i: (i, 0)),
            _const_spec((PATCH_LEN, HIDDEN)),
            _const_spec((1, HIDDEN)),
            _const_spec((HIDDEN, 1)),
            _const_spec((1, 1)),
            _const_spec((PATCH_LEN, D_MODEL)),
            _const_spec((PATCH_LEN, D_MODEL)),
            _const_spec((1, 1)),
            _const_spec((PATCH_NUM, D_MODEL)),
        ],
        out_specs=pl.BlockSpec((SB, PATCH_NUM, D_MODEL), lambda i: (i, 0, 0)),
        out_shape=jax.ShapeDtypeStruct((S, PATCH_NUM, D_MODEL), jnp.float32),
        compiler_params=pltpu.CompilerParams(
            dimension_semantics=("arbitrary",),
        ),
    )(sel_flat.reshape(S * PATCH_NUM, PATCH_LEN),
      org,
      W1_shf, b1_shf.reshape(1, HIDDEN), W2_shf, b2_shf.reshape(1, 1),
      W_reg, W_irr, alpha.reshape(1, 1), pe)
    return out
